# baseline (speedup 1.0000x reference)
import functools

import jax
import jax.numpy as jnp
from jax import lax
from jax.experimental import pallas as pl
from jax.experimental.pallas import tpu as pltpu
from jax.experimental.pallas import tpu_sc as plsc


def _sc_info():
    try:
        info = plsc.get_sparse_core_info()
        return info.num_cores, info.num_subcores
    except Exception:
        return 2, 16


def _pack_halves(lo16, hi16):
    ulo = pltpu.bitcast(lo16.astype(jnp.float32), jnp.uint32) >> 16
    uhi = pltpu.bitcast(hi16.astype(jnp.float32), jnp.uint32) & jnp.uint32(
        0xFFFF0000)
    return pltpu.bitcast(ulo | uhi, jnp.int32)


def _unpack_halves(packed):
    u = pltpu.bitcast(packed, jnp.uint32)
    lo = pltpu.bitcast(u << 16, jnp.float32)
    hi = pltpu.bitcast(u & jnp.uint32(0xFFFF0000), jnp.float32)
    return lo, hi


def _top2_from_logits(l):
    E = l.shape[1]
    iota = jax.lax.broadcasted_iota(jnp.int32, l.shape, 1)
    m1 = jnp.max(l, axis=1, keepdims=True)
    i1 = jnp.min(jnp.where(l == m1, iota, E), axis=1, keepdims=True)
    l2 = jnp.where(iota == i1, jnp.float32(-1e30), l)
    m2 = jnp.max(l2, axis=1, keepdims=True)
    i2 = jnp.min(jnp.where(l2 == m2, iota, E), axis=1, keepdims=True)
    w1 = 1.0 / (1.0 + jnp.exp(m2 - m1))
    w2 = 1.0 / (1.0 + jnp.exp(m1 - m2))
    sel = jnp.concatenate([i1, i2], axis=1)
    w = jnp.concatenate([w1, w2], axis=1)
    return sel, w


def _router_body(x_ref, wr_ref, sel_ref, w_ref, xp_ref):
    xb = x_ref[...].astype(jnp.bfloat16)
    logits = jax.lax.dot_general(
        xb, wr_ref[...].astype(jnp.bfloat16), (((1,), (1,)), ((), ())),
        preferred_element_type=jnp.float32)
    sel, w = _top2_from_logits(logits)
    sel_ref[...] = sel
    w_ref[...] = w
    d2 = xp_ref.shape[-1]
    xp_ref[...] = _pack_halves(xb[:, :d2], xb[:, d2:])


def _gmm_body(eid_ref, xs_ref, we_ref, be_ref, wt_ref, ys_ref):
    del eid_ref
    lo, hi = _unpack_halves(xs_ref[...])
    a = jnp.concatenate([lo, hi], axis=1).astype(jnp.bfloat16)
    h = jax.lax.dot_general(
        a, we_ref[0].astype(jnp.bfloat16), (((1,), (1,)), ((), ())),
        preferred_element_type=jnp.float32)
    y = ((h + be_ref[0]) * wt_ref[0]).astype(jnp.bfloat16)
    d2 = ys_ref.shape[-1]
    ys_ref[...] = _pack_halves(y[:, :d2], y[:, d2:])


def _add_body(g0_ref, g1_ref, out_ref):
    lo0, hi0 = _unpack_halves(g0_ref[...])
    lo1, hi1 = _unpack_halves(g1_ref[...])
    d2 = g0_ref.shape[-1]
    out_ref[:, :d2] = lo0 + lo1
    out_ref[:, d2:] = hi0 + hi1


def _sc_gather(x, idx, P, CH):
    N, D2 = x.shape
    nc, ns = _sc_info()
    NW = nc * ns
    H = P // 2
    rpw = H // NW
    nch = rpw // CH
    assert H * 2 == P and rpw % CH == 0 and nch % 2 == 0
    mesh = plsc.VectorSubcoreMesh(core_axis_name="c", subcore_axis_name="s",
                                  num_cores=nc, num_subcores=ns)

    @functools.partial(
        pl.kernel, mesh=mesh,
        out_type=jax.ShapeDtypeStruct((P, D2), x.dtype),
        scratch_types=[
            pltpu.VMEM((rpw,), jnp.int32),
            pltpu.VMEM((rpw,), jnp.int32),
            pltpu.VMEM((CH, D2), x.dtype),
            pltpu.VMEM((CH, D2), x.dtype),
            pltpu.VMEM((CH, D2), x.dtype),
            pltpu.VMEM((CH, D2), x.dtype),
            pltpu.SemaphoreType.DMA,
            pltpu.SemaphoreType.DMA,
            pltpu.SemaphoreType.DMA,
            pltpu.SemaphoreType.DMA,
        ],
    )
    def k(x_hbm, idx_hbm, out_hbm, i0_v, i1_v,
          a0, b0, a1, b1, sa0, sb0, sa1, sb1):
        wid = lax.axis_index("s") * nc + lax.axis_index("c")
        base = pl.multiple_of(wid * rpw, CH)
        pltpu.sync_copy(idx_hbm.at[pl.ds(base, rpw)], i0_v)
        pltpu.sync_copy(idx_hbm.at[pl.ds(H + base, rpw)], i1_v)
        pairs = ((a0, b0, sa0, sb0), (a1, b1, sa1, sb1))
        for b in range(2):
            A, Bb, sA, sB = pairs[b]
            pltpu.async_copy(x_hbm.at[i0_v.at[pl.ds(b * CH, CH)]], A, sA)
            pltpu.async_copy(x_hbm.at[i1_v.at[pl.ds(b * CH, CH)]], Bb, sB)

        def body(i, carry):
            for b in range(2):
                j = i * 2 + b
                A, Bb, sA, sB = pairs[b]
                off = pl.multiple_of(base + j * CH, CH)
                pltpu.make_async_copy(
                    x_hbm.at[i0_v.at[pl.ds(0, CH)]], A, sA).wait()
                pltpu.sync_copy(A, out_hbm.at[pl.ds(off, CH)])
                pltpu.make_async_copy(
                    x_hbm.at[i1_v.at[pl.ds(0, CH)]], Bb, sB).wait()
                pltpu.sync_copy(Bb, out_hbm.at[pl.ds(H + off, CH)])
                nj = j + 2

                @pl.when(nj < nch)
                def _():
                    pltpu.async_copy(
                        x_hbm.at[i0_v.at[pl.ds(nj * CH, CH)]], A, sA)
                    pltpu.async_copy(
                        x_hbm.at[i1_v.at[pl.ds(nj * CH, CH)]], Bb, sB)
            return carry

        lax.fori_loop(0, nch // 2, body, 0)

    return k(x, idx)


def _sc_gather2(ys, pos0, pos1, CH):
    P, D2 = ys.shape
    N = pos0.shape[0]
    nc, ns = _sc_info()
    NW = nc * ns
    rpw = N // NW
    nch = rpw // CH
    assert rpw % CH == 0 and nch % 2 == 0
    mesh = plsc.VectorSubcoreMesh(core_axis_name="c", subcore_axis_name="s",
                                  num_cores=nc, num_subcores=ns)

    @functools.partial(
        pl.kernel, mesh=mesh,
        out_type=(jax.ShapeDtypeStruct((N, D2), ys.dtype),
                  jax.ShapeDtypeStruct((N, D2), ys.dtype)),
        scratch_types=[
            pltpu.VMEM((rpw,), jnp.int32),
            pltpu.VMEM((rpw,), jnp.int32),
            pltpu.VMEM((CH, D2), ys.dtype),
            pltpu.VMEM((CH, D2), ys.dtype),
            pltpu.VMEM((CH, D2), ys.dtype),
            pltpu.VMEM((CH, D2), ys.dtype),
            pltpu.SemaphoreType.DMA,
            pltpu.SemaphoreType.DMA,
            pltpu.SemaphoreType.DMA,
            pltpu.SemaphoreType.DMA,
        ],
    )
    def k(ys_hbm, p0_hbm, p1_hbm, g0_hbm, g1_hbm, i0_v, i1_v,
          a0, b0, a1, b1, sa0, sb0, sa1, sb1):
        wid = lax.axis_index("s") * nc + lax.axis_index("c")
        base = pl.multiple_of(wid * rpw, CH)
        pltpu.sync_copy(p0_hbm.at[pl.ds(base, rpw)], i0_v)
        pltpu.sync_copy(p1_hbm.at[pl.ds(base, rpw)], i1_v)
        pairs = ((a0, b0, sa0, sb0), (a1, b1, sa1, sb1))
        for b in range(2):
            A, Bb, sA, sB = pairs[b]
            pltpu.async_copy(ys_hbm.at[i0_v.at[pl.ds(b * CH, CH)]], A, sA)
            pltpu.async_copy(ys_hbm.at[i1_v.at[pl.ds(b * CH, CH)]], Bb, sB)

        def body(i, carry):
            for b in range(2):
                j = i * 2 + b
                A, Bb, sA, sB = pairs[b]
                off = pl.multiple_of(base + j * CH, CH)
                pltpu.make_async_copy(
                    ys_hbm.at[i0_v.at[pl.ds(0, CH)]], A, sA).wait()
                pltpu.sync_copy(A, g0_hbm.at[pl.ds(off, CH)])
                pltpu.make_async_copy(
                    ys_hbm.at[i1_v.at[pl.ds(0, CH)]], Bb, sB).wait()
                pltpu.sync_copy(Bb, g1_hbm.at[pl.ds(off, CH)])
                nj = j + 2

                @pl.when(nj < nch)
                def _():
                    pltpu.async_copy(
                        ys_hbm.at[i0_v.at[pl.ds(nj * CH, CH)]], A, sA)
                    pltpu.async_copy(
                        ys_hbm.at[i1_v.at[pl.ds(nj * CH, CH)]], Bb, sB)
            return carry

        lax.fori_loop(0, nch // 2, body, 0)

    return k(ys, pos0, pos1)


def kernel(x, Wr, We, be):
    N, D = x.shape
    D2 = D // 2
    E = We.shape[0]
    TOPK = 2
    BLK = 256
    NBLK = (N * TOPK) // BLK + E
    P = NBLK * BLK
    BN = 512

    sel, topw, x_i32 = pl.pallas_call(
        _router_body,
        grid=(N // BN,),
        in_specs=[
            pl.BlockSpec((BN, D), lambda i: (i, 0)),
            pl.BlockSpec((E, D), lambda i: (0, 0)),
        ],
        out_specs=[
            pl.BlockSpec((BN, TOPK), lambda i: (i, 0)),
            pl.BlockSpec((BN, TOPK), lambda i: (i, 0)),
            pl.BlockSpec((BN, D2), lambda i: (i, 0)),
        ],
        out_shape=[
            jax.ShapeDtypeStruct((N, TOPK), jnp.int32),
            jax.ShapeDtypeStruct((N, TOPK), jnp.float32),
            jax.ShapeDtypeStruct((N, D2), jnp.int32),
        ],
    )(x, Wr)

    e_flat = sel.reshape(-1)
    w_flat = topw.reshape(-1)
    oh_t = (jnp.arange(E, dtype=jnp.int32)[:, None]
            == e_flat[None, :]).astype(jnp.int32)
    cum_t = jnp.cumsum(oh_t, axis=1)
    rank = jnp.sum(oh_t * cum_t, axis=0) - 1
    counts = cum_t[:, -1]
    padded = ((counts + BLK - 1) // BLK) * BLK
    cum_pad = jnp.cumsum(padded)
    pad_off = cum_pad - padded
    dest = (pad_off[e_flat] + rank).astype(jnp.int32)
    tok_flat = jnp.repeat(jnp.arange(N, dtype=jnp.int32), TOPK)
    tok_padded = jnp.zeros((P,), jnp.int32).at[dest].set(tok_flat)
    wt_padded = jnp.zeros((P,), jnp.float32).at[dest].set(w_flat)
    eid = jnp.clip(
        jnp.searchsorted(cum_pad, jnp.arange(NBLK) * BLK, side="right"),
        0, E - 1).astype(jnp.int32)
    pos0 = dest[0::2]
    pos1 = dest[1::2]

    xs_i32 = _sc_gather(x_i32, tok_padded, P, CH=24)

    be3 = be.reshape(E, 1, D)
    wt3 = wt_padded.reshape(NBLK, BLK, 1)
    grid_spec = pltpu.PrefetchScalarGridSpec(
        num_scalar_prefetch=1,
        grid=(NBLK,),
        in_specs=[
            pl.BlockSpec((BLK, D2), lambda i, eid_r: (i, 0)),
            pl.BlockSpec((1, D, D), lambda i, eid_r: (eid_r[i], 0, 0)),
            pl.BlockSpec((1, 1, D), lambda i, eid_r: (eid_r[i], 0, 0)),
            pl.BlockSpec((1, BLK, 1), lambda i, eid_r: (i, 0, 0)),
        ],
        out_specs=pl.BlockSpec((BLK, D2), lambda i, eid_r: (i, 0)),
    )
    ys_i32 = pl.pallas_call(
        _gmm_body,
        grid_spec=grid_spec,
        out_shape=jax.ShapeDtypeStruct((P, D2), jnp.int32),
    )(eid, xs_i32, We, be3, wt3)

    g0, g1 = _sc_gather2(ys_i32, pos0, pos1, CH=16)

    out = pl.pallas_call(
        _add_body,
        grid=(N // BN,),
        in_specs=[
            pl.BlockSpec((BN, D2), lambda i: (i, 0)),
            pl.BlockSpec((BN, D2), lambda i: (i, 0)),
        ],
        out_specs=pl.BlockSpec((BN, D), lambda i: (i, 0)),
        out_shape=jax.ShapeDtypeStruct((N, D), jnp.float32),
    )(g0, g1)
    return out


# speedup vs baseline: 1.0075x; 1.0075x over previous
import functools

import jax
import jax.numpy as jnp
from jax import lax
from jax.experimental import pallas as pl
from jax.experimental.pallas import tpu as pltpu
from jax.experimental.pallas import tpu_sc as plsc


def _sc_info():
    try:
        info = plsc.get_sparse_core_info()
        return info.num_cores, info.num_subcores
    except Exception:
        return 2, 16


def _pack_halves(lo16, hi16):
    ulo = pltpu.bitcast(lo16.astype(jnp.float32), jnp.uint32) >> 16
    uhi = pltpu.bitcast(hi16.astype(jnp.float32), jnp.uint32) & jnp.uint32(
        0xFFFF0000)
    return pltpu.bitcast(ulo | uhi, jnp.int32)


def _unpack_halves(packed):
    u = pltpu.bitcast(packed, jnp.uint32)
    lo = pltpu.bitcast(u << 16, jnp.float32)
    hi = pltpu.bitcast(u & jnp.uint32(0xFFFF0000), jnp.float32)
    return lo, hi


def _top2_from_logits(l):
    E = l.shape[1]
    iota = jax.lax.broadcasted_iota(jnp.int32, l.shape, 1)
    m1 = jnp.max(l, axis=1, keepdims=True)
    i1 = jnp.min(jnp.where(l == m1, iota, E), axis=1, keepdims=True)
    l2 = jnp.where(iota == i1, jnp.float32(-1e30), l)
    m2 = jnp.max(l2, axis=1, keepdims=True)
    i2 = jnp.min(jnp.where(l2 == m2, iota, E), axis=1, keepdims=True)
    w1 = 1.0 / (1.0 + jnp.exp(m2 - m1))
    w2 = 1.0 / (1.0 + jnp.exp(m1 - m2))
    sel = jnp.concatenate([i1, i2], axis=1)
    w = jnp.concatenate([w1, w2], axis=1)
    return sel, w


def _router_body(x_ref, wr_ref, sel_ref, w_ref, xp_ref):
    xb = x_ref[...].astype(jnp.bfloat16)
    logits = jax.lax.dot_general(
        xb, wr_ref[...].astype(jnp.bfloat16), (((1,), (1,)), ((), ())),
        preferred_element_type=jnp.float32)
    sel, w = _top2_from_logits(logits)
    sel_ref[...] = sel
    w_ref[...] = w
    d2 = xp_ref.shape[-1]
    xp_ref[...] = _pack_halves(xb[:, :d2], xb[:, d2:])


def _gmm_body(eid_ref, xs_ref, we_ref, be_ref, ys_ref):
    del eid_ref
    lo, hi = _unpack_halves(xs_ref[...])
    a = jnp.concatenate([lo, hi], axis=1).astype(jnp.bfloat16)
    h = jax.lax.dot_general(
        a, we_ref[0].astype(jnp.bfloat16), (((1,), (1,)), ((), ())),
        preferred_element_type=jnp.float32)
    y = (h + be_ref[0]).astype(jnp.bfloat16)
    d2 = ys_ref.shape[-1]
    ys_ref[...] = _pack_halves(y[:, :d2], y[:, d2:])


def _add_body(g0_ref, g1_ref, w_ref, out_ref):
    lo0, hi0 = _unpack_halves(g0_ref[...])
    lo1, hi1 = _unpack_halves(g1_ref[...])
    w0 = w_ref[:, 0:1]
    w1 = w_ref[:, 1:2]
    d2 = g0_ref.shape[-1]
    out_ref[:, :d2] = w0 * lo0 + w1 * lo1
    out_ref[:, d2:] = w0 * hi0 + w1 * hi1


def _sc_gather(x, idx, P, CH):
    N, D2 = x.shape
    nc, ns = _sc_info()
    NW = nc * ns
    H = P // 2
    rpw = H // NW
    nch = rpw // CH
    assert H * 2 == P and rpw % CH == 0 and nch % 2 == 0
    mesh = plsc.VectorSubcoreMesh(core_axis_name="c", subcore_axis_name="s",
                                  num_cores=nc, num_subcores=ns)

    @functools.partial(
        pl.kernel, mesh=mesh,
        out_type=jax.ShapeDtypeStruct((P, D2), x.dtype),
        scratch_types=[
            pltpu.VMEM((rpw,), jnp.int32),
            pltpu.VMEM((rpw,), jnp.int32),
            pltpu.VMEM((CH, D2), x.dtype),
            pltpu.VMEM((CH, D2), x.dtype),
            pltpu.VMEM((CH, D2), x.dtype),
            pltpu.VMEM((CH, D2), x.dtype),
            pltpu.SemaphoreType.DMA,
            pltpu.SemaphoreType.DMA,
            pltpu.SemaphoreType.DMA,
            pltpu.SemaphoreType.DMA,
        ],
    )
    def k(x_hbm, idx_hbm, out_hbm, i0_v, i1_v,
          a0, b0, a1, b1, sa0, sb0, sa1, sb1):
        wid = lax.axis_index("s") * nc + lax.axis_index("c")
        base = pl.multiple_of(wid * rpw, CH)
        pltpu.sync_copy(idx_hbm.at[pl.ds(base, rpw)], i0_v)
        pltpu.sync_copy(idx_hbm.at[pl.ds(H + base, rpw)], i1_v)
        pairs = ((a0, b0, sa0, sb0), (a1, b1, sa1, sb1))
        for b in range(2):
            A, Bb, sA, sB = pairs[b]
            pltpu.async_copy(x_hbm.at[i0_v.at[pl.ds(b * CH, CH)]], A, sA)
            pltpu.async_copy(x_hbm.at[i1_v.at[pl.ds(b * CH, CH)]], Bb, sB)

        def body(i, carry):
            for b in range(2):
                j = i * 2 + b
                A, Bb, sA, sB = pairs[b]
                off = pl.multiple_of(base + j * CH, CH)
                pltpu.make_async_copy(
                    x_hbm.at[i0_v.at[pl.ds(0, CH)]], A, sA).wait()
                pltpu.sync_copy(A, out_hbm.at[pl.ds(off, CH)])
                pltpu.make_async_copy(
                    x_hbm.at[i1_v.at[pl.ds(0, CH)]], Bb, sB).wait()
                pltpu.sync_copy(Bb, out_hbm.at[pl.ds(H + off, CH)])
                nj = j + 2

                @pl.when(nj < nch)
                def _():
                    pltpu.async_copy(
                        x_hbm.at[i0_v.at[pl.ds(nj * CH, CH)]], A, sA)
                    pltpu.async_copy(
                        x_hbm.at[i1_v.at[pl.ds(nj * CH, CH)]], Bb, sB)
            return carry

        lax.fori_loop(0, nch // 2, body, 0)

    return k(x, idx)


def _sc_gather2(ys, pos0, pos1, CH):
    P, D2 = ys.shape
    N = pos0.shape[0]
    nc, ns = _sc_info()
    NW = nc * ns
    rpw = N // NW
    nch = rpw // CH
    assert rpw % CH == 0 and nch % 2 == 0
    mesh = plsc.VectorSubcoreMesh(core_axis_name="c", subcore_axis_name="s",
                                  num_cores=nc, num_subcores=ns)

    @functools.partial(
        pl.kernel, mesh=mesh,
        out_type=(jax.ShapeDtypeStruct((N, D2), ys.dtype),
                  jax.ShapeDtypeStruct((N, D2), ys.dtype)),
        scratch_types=[
            pltpu.VMEM((rpw,), jnp.int32),
            pltpu.VMEM((rpw,), jnp.int32),
            pltpu.VMEM((CH, D2), ys.dtype),
            pltpu.VMEM((CH, D2), ys.dtype),
            pltpu.VMEM((CH, D2), ys.dtype),
            pltpu.VMEM((CH, D2), ys.dtype),
            pltpu.SemaphoreType.DMA,
            pltpu.SemaphoreType.DMA,
            pltpu.SemaphoreType.DMA,
            pltpu.SemaphoreType.DMA,
        ],
    )
    def k(ys_hbm, p0_hbm, p1_hbm, g0_hbm, g1_hbm, i0_v, i1_v,
          a0, b0, a1, b1, sa0, sb0, sa1, sb1):
        wid = lax.axis_index("s") * nc + lax.axis_index("c")
        base = pl.multiple_of(wid * rpw, CH)
        pltpu.sync_copy(p0_hbm.at[pl.ds(base, rpw)], i0_v)
        pltpu.sync_copy(p1_hbm.at[pl.ds(base, rpw)], i1_v)
        pairs = ((a0, b0, sa0, sb0), (a1, b1, sa1, sb1))
        for b in range(2):
            A, Bb, sA, sB = pairs[b]
            pltpu.async_copy(ys_hbm.at[i0_v.at[pl.ds(b * CH, CH)]], A, sA)
            pltpu.async_copy(ys_hbm.at[i1_v.at[pl.ds(b * CH, CH)]], Bb, sB)

        def body(i, carry):
            for b in range(2):
                j = i * 2 + b
                A, Bb, sA, sB = pairs[b]
                off = pl.multiple_of(base + j * CH, CH)
                pltpu.make_async_copy(
                    ys_hbm.at[i0_v.at[pl.ds(0, CH)]], A, sA).wait()
                pltpu.sync_copy(A, g0_hbm.at[pl.ds(off, CH)])
                pltpu.make_async_copy(
                    ys_hbm.at[i1_v.at[pl.ds(0, CH)]], Bb, sB).wait()
                pltpu.sync_copy(Bb, g1_hbm.at[pl.ds(off, CH)])
                nj = j + 2

                @pl.when(nj < nch)
                def _():
                    pltpu.async_copy(
                        ys_hbm.at[i0_v.at[pl.ds(nj * CH, CH)]], A, sA)
                    pltpu.async_copy(
                        ys_hbm.at[i1_v.at[pl.ds(nj * CH, CH)]], Bb, sB)
            return carry

        lax.fori_loop(0, nch // 2, body, 0)

    return k(ys, pos0, pos1)


def kernel(x, Wr, We, be):
    N, D = x.shape
    D2 = D // 2
    E = We.shape[0]
    TOPK = 2
    BLK = 256
    NBLK = (N * TOPK) // BLK + E
    P = NBLK * BLK
    BN = 512

    sel, topw, x_i32 = pl.pallas_call(
        _router_body,
        grid=(N // BN,),
        in_specs=[
            pl.BlockSpec((BN, D), lambda i: (i, 0)),
            pl.BlockSpec((E, D), lambda i: (0, 0)),
        ],
        out_specs=[
            pl.BlockSpec((BN, TOPK), lambda i: (i, 0)),
            pl.BlockSpec((BN, TOPK), lambda i: (i, 0)),
            pl.BlockSpec((BN, D2), lambda i: (i, 0)),
        ],
        out_shape=[
            jax.ShapeDtypeStruct((N, TOPK), jnp.int32),
            jax.ShapeDtypeStruct((N, TOPK), jnp.float32),
            jax.ShapeDtypeStruct((N, D2), jnp.int32),
        ],
    )(x, Wr)

    e_flat = sel.reshape(-1)
    w_flat = topw.reshape(-1)
    oh_t = (jnp.arange(E, dtype=jnp.int32)[:, None]
            == e_flat[None, :]).astype(jnp.int32)
    cum_t = jnp.cumsum(oh_t, axis=1)
    rank = jnp.sum(oh_t * cum_t, axis=0) - 1
    counts = cum_t[:, -1]
    padded = ((counts + BLK - 1) // BLK) * BLK
    cum_pad = jnp.cumsum(padded)
    pad_off = cum_pad - padded
    dest = (pad_off[e_flat] + rank).astype(jnp.int32)
    tok_flat = jnp.repeat(jnp.arange(N, dtype=jnp.int32), TOPK)
    tok_padded = jnp.zeros((P,), jnp.int32).at[dest].set(tok_flat)
    eid = jnp.minimum(
        jnp.sum((jnp.arange(NBLK, dtype=jnp.int32) * BLK)[:, None]
                >= cum_pad[None, :].astype(jnp.int32), axis=1),
        E - 1).astype(jnp.int32)
    pos0 = dest[0::2]
    pos1 = dest[1::2]

    xs_i32 = _sc_gather(x_i32, tok_padded, P, CH=24)

    be3 = be.reshape(E, 1, D)
    grid_spec = pltpu.PrefetchScalarGridSpec(
        num_scalar_prefetch=1,
        grid=(NBLK,),
        in_specs=[
            pl.BlockSpec((BLK, D2), lambda i, eid_r: (i, 0)),
            pl.BlockSpec((1, D, D), lambda i, eid_r: (eid_r[i], 0, 0)),
            pl.BlockSpec((1, 1, D), lambda i, eid_r: (eid_r[i], 0, 0)),
        ],
        out_specs=pl.BlockSpec((BLK, D2), lambda i, eid_r: (i, 0)),
    )
    ys_i32 = pl.pallas_call(
        _gmm_body,
        grid_spec=grid_spec,
        out_shape=jax.ShapeDtypeStruct((P, D2), jnp.int32),
    )(eid, xs_i32, We, be3)

    g0, g1 = _sc_gather2(ys_i32, pos0, pos1, CH=16)

    out = pl.pallas_call(
        _add_body,
        grid=(N // BN,),
        in_specs=[
            pl.BlockSpec((BN, D2), lambda i: (i, 0)),
            pl.BlockSpec((BN, D2), lambda i: (i, 0)),
            pl.BlockSpec((BN, TOPK), lambda i: (i, 0)),
        ],
        out_specs=pl.BlockSpec((BN, D), lambda i: (i, 0)),
        out_shape=jax.ShapeDtypeStruct((N, D), jnp.float32),
    )(g0, g1, topw)
    return out
